# Initial kernel scaffold; baseline (speedup 1.0000x reference)
import jax
import jax.numpy as jnp
from jax.experimental import pallas as pl


def kernel(x, edge_index, batch, W1, att_src1, att_dst1, bias1, W2, att_src2, att_dst2, bias2, W3, att_src3, att_dst3, bias3, lin1_W, lin1_b, lin2_W, lin2_b):
    raise NotImplementedError("write your pallas kernel here")


# speedup vs baseline: 59.0261x; 59.0261x over previous
import functools

import jax
import jax.numpy as jnp
from jax import lax
from jax.experimental import pallas as pl
from jax.experimental.pallas import tpu as pltpu
from jax.experimental.pallas import tpu_sc as plsc

N_NODES = 10000
N_EDGES = 320000
IN_CH = 128
HID = 64
NUM_CLASSES = 3
NUM_GRAPHS = 128

N_PAD = 10240
N_ROWS = N_PAD // 16
NSC = 2
NTILES = 16
NW = NSC * NTILES
E_PER_TILE1 = N_EDGES // NTILES
P1C = 4000
B2 = 128
NB2 = N_EDGES // B2
NB_MAIN = NB2 // NW


def _pre1_body(x_ref, w_ref, asrc_ref, adst_ref, h_ref, es_ref, ed_ref):
    h = jnp.dot(x_ref[...], w_ref[...], preferred_element_type=jnp.float32)
    h_ref[...] = h
    es_ref[...] = jnp.dot(h, asrc_ref[...], preferred_element_type=jnp.float32)
    ed_ref[...] = jnp.dot(h, adst_ref[...], preferred_element_type=jnp.float32)


def _pre23_body(op_ref, b_ref, w_ref, asrc_ref, adst_ref, h_ref, es_ref, ed_ref):
    xl = jnp.maximum(op_ref[0, :N_NODES] + op_ref[1, :N_NODES] + b_ref[...],
                     0.0)
    h = jnp.dot(xl, w_ref[...], preferred_element_type=jnp.float32)
    h_ref[...] = h
    es_ref[...] = jnp.dot(h, asrc_ref[...], preferred_element_type=jnp.float32)
    ed_ref[...] = jnp.dot(h, adst_ref[...], preferred_element_type=jnp.float32)


def _post_body(op_ref, b_ref, batch_ref, l1w_ref, l1b_ref, l2w_ref, l2b_ref,
               out_ref):
    h = jnp.maximum(op_ref[0, :N_NODES] + op_ref[1, :N_NODES] + b_ref[...],
                    0.0)
    gid = lax.broadcasted_iota(jnp.int32, (1, NUM_GRAPHS), 1)
    oh = (batch_ref[...] == gid).astype(jnp.float32)
    sums = lax.dot_general(oh, h, (((0,), (0,)), ((), ())),
                           preferred_element_type=jnp.float32)
    ones = jnp.ones((N_NODES, 1), jnp.float32)
    counts = lax.dot_general(oh, ones, (((0,), (0,)), ((), ())),
                             preferred_element_type=jnp.float32)
    pooled = sums / jnp.maximum(counts, 1.0)
    g = jnp.maximum(
        jnp.dot(pooled, l1w_ref[...], preferred_element_type=jnp.float32)
        + l1b_ref[...], 0.0)
    out_ref[...] = (
        jnp.dot(g, l2w_ref[...], preferred_element_type=jnp.float32)
        + l2b_ref[...])


def _pre1(x, w, asrc, adst):
    return pl.pallas_call(
        _pre1_body,
        out_shape=(
            jax.ShapeDtypeStruct((N_NODES, HID), jnp.float32),
            jax.ShapeDtypeStruct((N_NODES, 1), jnp.float32),
            jax.ShapeDtypeStruct((N_NODES, 1), jnp.float32),
        ),
    )(x, w, asrc, adst)


def _pre23(op, b, w, asrc, adst):
    return pl.pallas_call(
        _pre23_body,
        out_shape=(
            jax.ShapeDtypeStruct((N_NODES, HID), jnp.float32),
            jax.ShapeDtypeStruct((N_NODES, 1), jnp.float32),
            jax.ShapeDtypeStruct((N_NODES, 1), jnp.float32),
        ),
    )(op, b, w, asrc, adst)


def _post(op, b, batch2d, l1w, l1b, l2w, l2b):
    return pl.pallas_call(
        _post_body,
        out_shape=jax.ShapeDtypeStruct((NUM_GRAPHS, NUM_CLASSES), jnp.float32),
    )(op, b, batch2d, l1w, l1b, l2w, l2b)


def _edge_body(h_hbm, es_hbm, ed_hbm, ei_hbm, out_hbm,
               es_v, ed_v, den_v, srcA_v, dstA_v, srcB_v, dstB_v,
               src2_v, dst2_v, srcb_v, dstb_v, rows_a, rows_b, idxi_v,
               shared_out, shared_den, sem_i, sem_p1, sem_ga, sem_gb,
               sem_sa, sem_sb):
    c = lax.axis_index("c")
    s = lax.axis_index("s")
    wid = c * NTILES + s

    zero16 = jnp.zeros((16,), jnp.float32)

    e_stage = [
        pltpu.async_copy(es_hbm, es_v, sem_i),
        pltpu.async_copy(ed_hbm, ed_v, sem_i),
    ]

    def _zden(i, carry):
        den_v[i] = zero16
        return carry
    lax.fori_loop(0, N_ROWS, _zden, 0)

    def _zrow(i, carry):
        for f in range(HID // 16):
            rows_a[i, pl.ds(f * 16, 16)] = zero16
        return carry
    lax.fori_loop(0, B2, _zrow, 0)

    for j in range(5):
        for v in range(8):
            idxi_v[j, pl.ds(v * 16, 16)] = (
                lax.iota(jnp.int32, 16) + (j * 128 + v * 16))

    @pl.when(s == 0)
    def _():
        pltpu.sync_copy(den_v, shared_den)
    for j in range(5):
        pltpu.sync_copy(rows_a, shared_out.at[pl.ds((s * 5 + j) * B2, B2)])

    for d in e_stage:
        d.wait()

    def _mx(i, carry):
        a, b = carry
        return jnp.maximum(a, es_v[i]), jnp.maximum(b, ed_v[i])
    m_es, m_ed = lax.fori_loop(
        0, N_ROWS, _mx,
        (jnp.full((16,), -1e30, jnp.float32), jnp.full((16,), -1e30, jnp.float32)))
    def _lanemax(v):
        k_sorted, _ = plsc.sort_key_val(v, v, descending=True)
        return k_sorted[0]
    zmax = _lanemax(m_es) + _lanemax(m_ed)
    m_glob = jnp.where(zmax > 0, zmax, 0.2 * zmax)

    off2 = wid * (NB_MAIN * B2)
    src2_stage = pltpu.async_copy(
        ei_hbm.at[0, pl.ds(off2, NB_MAIN * B2)], src2_v, sem_ga)

    p1_pairs = ((srcA_v, dstA_v), (srcB_v, dstB_v))

    def _fire_p1(chunk):
        off1 = s * E_PER_TILE1 + chunk * P1C
        pair = p1_pairs[chunk % 2]
        pltpu.async_copy(ei_hbm.at[0, pl.ds(off1, P1C)], pair[0], sem_p1)
        pltpu.async_copy(ei_hbm.at[1, pl.ds(off1, P1C)], pair[1], sem_p1)

    _fire_p1(0)

    plsc.subcore_barrier()

    src2_stage.wait()
    pltpu.async_copy(h_hbm.at[src2_v.at[pl.ds(0, B2)]], rows_a, sem_ga)

    def _p1(t, carry, sbuf, dbuf):
        s_idx = sbuf[pl.ds(t * 16, 16)]
        d_idx = dbuf[pl.ds(t * 16, 16)]
        es = plsc.load_gather(es_v, [s_idx >> 4, s_idx & 15])
        ed = plsc.load_gather(ed_v, [d_idx >> 4, d_idx & 15])
        z = es + ed
        logit = jnp.where(z > 0, z, 0.2 * z)
        p = jnp.exp(logit - m_glob)
        plsc.addupdate_scatter(den_v, [d_idx >> 4, d_idx & 15], p)
        return carry
    for chunk in range(E_PER_TILE1 // P1C):
        sbuf, dbuf = p1_pairs[chunk % 2]
        pltpu.make_async_copy(ei_hbm.at[0, pl.ds(0, P1C)], sbuf, sem_p1).wait()
        pltpu.make_async_copy(ei_hbm.at[1, pl.ds(0, P1C)], dbuf, sem_p1).wait()
        if chunk + 1 < E_PER_TILE1 // P1C:
            _fire_p1(chunk + 1)
        lax.fori_loop(0, P1C // 16,
                      functools.partial(_p1, sbuf=sbuf, dbuf=dbuf), 0)

    for j in range(5):
        pltpu.sync_copy(den_v.at[pl.ds(j * 128, 128)],
                        shared_den.at[idxi_v.at[j]], add=True)
    plsc.subcore_barrier()
    pltpu.sync_copy(shared_den, den_v)

    part_rows = NB_MAIN // 3
    for part in range(3):
        pltpu.sync_copy(
            ei_hbm.at[1, pl.ds(off2 + part * part_rows * B2, part_rows * B2)],
            dstA_v.at[pl.ds(0, part_rows * B2)])

        def _mkrows(t, carry):
            for v in range(B2 // 16):
                dst2_v[part * part_rows + t, pl.ds(v * 16, 16)] = (
                    dstA_v[pl.ds(t * B2 + v * 16, 16)])
            return carry
        lax.fori_loop(0, part_rows, _mkrows, 0)

    def _scale(buf, s_flat, s_base, d2_ref, d_row):
        for j in range(B2 // 16):
            s_idx = s_flat[pl.ds(s_base + j * 16, 16)]
            d_idx = d2_ref[d_row, pl.ds(j * 16, 16)]
            es = plsc.load_gather(es_v, [s_idx >> 4, s_idx & 15])
            ed = plsc.load_gather(ed_v, [d_idx >> 4, d_idx & 15])
            z = es + ed
            logit = jnp.where(z > 0, z, 0.2 * z)
            p = jnp.exp(logit - m_glob)
            den = plsc.load_gather(den_v, [d_idx >> 4, d_idx & 15])
            alpha16 = p / (den + 1e-16)
            for k in range(16):
                a = alpha16[k]
                r = j * 16 + k
                for f in range(HID // 16):
                    sl = pl.ds(f * 16, 16)
                    buf[r, sl] = buf[r, sl] * a

    def _gather(t, buf, sem):
        return pltpu.async_copy(h_hbm.at[src2_v.at[pl.ds(t * B2, B2)]],
                                buf, sem)

    def _wait_gather(buf, sem):
        pltpu.make_async_copy(h_hbm.at[src2_v.at[pl.ds(0, B2)]], buf,
                              sem).wait()

    def _scatter(t, buf, sem):
        return pltpu.async_copy(buf, shared_out.at[dst2_v.at[t]], sem,
                                add=True)

    def _wait_scatter(buf, sem):
        pltpu.make_async_copy(buf, shared_out.at[dst2_v.at[0]], sem).wait()

    def _p2(i, carry):
        t0 = 2 * i
        t1 = t0 + 1
        _wait_gather(rows_a, sem_ga)

        @pl.when(i > 0)
        def _():
            _wait_scatter(rows_b, sem_sb)
        _gather(t1, rows_b, sem_gb)
        _scale(rows_a, src2_v, t0 * B2, dst2_v, t0)
        _scatter(t0, rows_a, sem_sa)

        _wait_gather(rows_b, sem_gb)
        _wait_scatter(rows_a, sem_sa)

        @pl.when(i < NB_MAIN // 2 - 1)
        def _():
            _gather(t0 + 2, rows_a, sem_ga)
        _scale(rows_b, src2_v, t1 * B2, dst2_v, t1)
        _scatter(t1, rows_b, sem_sb)
        return carry
    lax.fori_loop(0, NB_MAIN // 2, _p2, 0)
    _wait_scatter(rows_b, sem_sb)

    @pl.when(wid < NB2 - NW * NB_MAIN)
    def _():
        tb = NW * NB_MAIN + wid
        off = tb * B2
        pltpu.sync_copy(ei_hbm.at[0, pl.ds(off, B2)], srcb_v)
        pltpu.sync_copy(ei_hbm.at[1, pl.ds(off, B2)], dstb_v.at[0])
        pltpu.async_copy(h_hbm.at[srcb_v], rows_a, sem_ga).wait()
        _scale(rows_a, srcb_v, 0, dstb_v, 0)
        pltpu.async_copy(rows_a, shared_out.at[dstb_v.at[0]], sem_sa,
                         add=True).wait()

    plsc.subcore_barrier()

    rows_per_tile = N_PAD // NTILES
    pltpu.sync_copy(shared_out.at[pl.ds(s * rows_per_tile, rows_per_tile)],
                    out_hbm.at[c, pl.ds(s * rows_per_tile, rows_per_tile)])


def _edge(h, es640, ed640, ei):
    mesh = plsc.VectorSubcoreMesh(core_axis_name="c", subcore_axis_name="s",
                                  num_cores=NSC, num_subcores=NTILES)
    fn = pl.kernel(
        _edge_body,
        out_type=jax.ShapeDtypeStruct((NSC, N_PAD, HID), jnp.float32),
        mesh=mesh,
        compiler_params=pltpu.CompilerParams(needs_layout_passes=False,
                                             use_tc_tiling_on_sc=False),
        scratch_types=[
            pltpu.VMEM((N_ROWS, 16), jnp.float32),
            pltpu.VMEM((N_ROWS, 16), jnp.float32),
            pltpu.VMEM((N_ROWS, 16), jnp.float32),
            pltpu.VMEM((P1C,), jnp.int32),
            pltpu.VMEM((P1C,), jnp.int32),
            pltpu.VMEM((P1C,), jnp.int32),
            pltpu.VMEM((P1C,), jnp.int32),
            pltpu.VMEM((NB_MAIN * B2,), jnp.int32),
            pltpu.VMEM((NB_MAIN, B2), jnp.int32),
            pltpu.VMEM((B2,), jnp.int32),
            pltpu.VMEM((1, B2), jnp.int32),
            pltpu.VMEM((B2, HID), jnp.float32),
            pltpu.VMEM((B2, HID), jnp.float32),
            pltpu.VMEM((5, 128), jnp.int32),
            pltpu.VMEM_SHARED((N_PAD, HID), jnp.float32),
            pltpu.VMEM_SHARED((N_ROWS, 16), jnp.float32),
            pltpu.SemaphoreType.DMA,
            pltpu.SemaphoreType.DMA,
            pltpu.SemaphoreType.DMA,
            pltpu.SemaphoreType.DMA,
            pltpu.SemaphoreType.DMA,
            pltpu.SemaphoreType.DMA,
        ],
    )
    return fn(h, es640, ed640, ei)


def _pad640(e):
    return jnp.pad(e[:, 0], (0, N_PAD - N_NODES)).reshape(N_ROWS, 16)


def kernel(x, edge_index, batch,
           W1, att_src1, att_dst1, bias1,
           W2, att_src2, att_dst2, bias2,
           W3, att_src3, att_dst3, bias3,
           lin1_W, lin1_b, lin2_W, lin2_b):
    h, es, ed = _pre1(x, W1, att_src1[:, None], att_dst1[:, None])
    op = _edge(h, _pad640(es), _pad640(ed), edge_index)

    h, es, ed = _pre23(op, bias1[None, :], W2, att_src2[:, None],
                       att_dst2[:, None])
    op = _edge(h, _pad640(es), _pad640(ed), edge_index)

    h, es, ed = _pre23(op, bias2[None, :], W3, att_src3[:, None],
                       att_dst3[:, None])
    op = _edge(h, _pad640(es), _pad640(ed), edge_index)

    return _post(op, bias3[None, :], batch[:, None].astype(jnp.int32),
                 lin1_W, lin1_b[None, :], lin2_W, lin2_b[None, :])
